# baseline (speedup 1.0000x reference)
import functools

import jax
import jax.numpy as jnp
from jax import lax
from jax.experimental import pallas as pl
from jax.experimental.pallas import tpu as pltpu
from jax.experimental.pallas import tpu_sc as plsc

TM = 128
TN = 4096
CHUNK = 32


def _sc_scatter(x, si0, si1, M):
    ntokens, K = x.shape
    info = plsc.get_sparse_core_info()
    NW = info.num_cores * info.num_subcores
    tpw = ntokens // NW
    nchunks = tpw // CHUNK
    mesh = plsc.VectorSubcoreMesh(core_axis_name="c", subcore_axis_name="s")

    @functools.partial(
        pl.kernel,
        mesh=mesh,
        out_type=jax.ShapeDtypeStruct((M, K), jnp.float32),
        scratch_types=[
            pltpu.VMEM((CHUNK, K), jnp.float32),
            pltpu.VMEM((CHUNK,), jnp.int32),
            pltpu.VMEM((CHUNK,), jnp.int32),
            pltpu.SemaphoreType.DMA,
        ],
    )
    def scatter_k(x_hbm, si0_hbm, si1_hbm, out_hbm, rows_v, i0_v, i1_v, sem):
        wid = lax.axis_index("s") * info.num_cores + lax.axis_index("c")
        base = wid * tpw
        for c in range(nchunks):
            t0 = base + c * CHUNK
            pltpu.sync_copy(x_hbm.at[pl.ds(t0, CHUNK)], rows_v)
            pltpu.sync_copy(si0_hbm.at[pl.ds(t0, CHUNK)], i0_v)
            pltpu.sync_copy(si1_hbm.at[pl.ds(t0, CHUNK)], i1_v)
            cp0 = pltpu.async_copy(rows_v, out_hbm.at[i0_v], sem)
            cp1 = pltpu.async_copy(rows_v, out_hbm.at[i1_v], sem)
            cp0.wait()
            cp1.wait()

    return scatter_k(x, si0, si1)


def _tile_metadata(splits, M, E, G):
    NT = M // TM
    ends = jnp.cumsum(splits)
    starts = ends - splits
    t_lo = starts // TM
    t_hi = jnp.where(splits > 0, (ends - 1) // TM, t_lo)
    tiles = jnp.where(splits > 0, t_hi - t_lo + 1, 0)
    bounds = jnp.cumsum(tiles)
    p = jnp.arange(G, dtype=jnp.int32)
    e_p = jnp.sum((p[:, None] >= bounds[None, :]).astype(jnp.int32), axis=1)
    valid = e_p < E
    e_c = jnp.minimum(e_p, E - 1)
    prev = jnp.where(e_c > 0, jnp.take(bounds, e_c - 1, mode="clip"), 0)
    t_p = jnp.where(valid, jnp.take(t_lo, e_c) + (p - prev), NT - 1)
    ps = jnp.where(valid, jnp.maximum(jnp.take(starts, e_c), t_p * TM), 1)
    pe = jnp.where(valid, jnp.minimum(jnp.take(ends, e_c), (t_p + 1) * TM), 0)
    first = jnp.concatenate(
        [jnp.ones((1,), jnp.int32), (t_p[1:] != t_p[:-1]).astype(jnp.int32)]
    )
    return jnp.stack([e_c, t_p, ps, pe, first]).astype(jnp.int32)


def _gemm_body(meta_ref, x_ref, w_ref, b_ref, out_ref):
    p = pl.program_id(1)
    t = meta_ref[1, p]
    ps = meta_ref[2, p]
    pe = meta_ref[3, p]
    first = meta_ref[4, p]

    def contrib():
        acc = lax.dot_general(
            x_ref[...], w_ref[0],
            (((1,), (1,)), ((), ())),
            preferred_element_type=jnp.float32,
        )
        acc = acc + b_ref[0]
        rows = t * TM + lax.broadcasted_iota(jnp.int32, (TM, 1), 0)
        return jnp.where((rows >= ps) & (rows < pe), acc, 0.0)

    @pl.when(first == 1)
    def _():
        out_ref[...] = contrib()

    @pl.when((first == 0) & (pe > ps))
    def _():
        out_ref[...] = out_ref[...] + contrib()


def kernel(input, weights, bias, splits_gpu, scatter_index):
    ntokens, topk = scatter_index.shape
    K = input.shape[1]
    E, N, _ = weights.shape
    M = ntokens * topk
    G = M // TM + E - 1

    si = scatter_index.astype(jnp.int32)
    x_g = _sc_scatter(input, si[:, 0], si[:, 1], M)

    meta = _tile_metadata(splits_gpu.astype(jnp.int32), M, E, G)

    grid_spec = pltpu.PrefetchScalarGridSpec(
        num_scalar_prefetch=1,
        grid=(N // TN, G),
        in_specs=[
            pl.BlockSpec((TM, K), lambda n, p, m: (m[1, p], 0)),
            pl.BlockSpec((1, TN, K), lambda n, p, m: (m[0, p], n, 0)),
            pl.BlockSpec((1, 1, TN), lambda n, p, m: (m[0, p], 0, n)),
        ],
        out_specs=pl.BlockSpec((TM, TN), lambda n, p, m: (m[1, p], n)),
    )
    return pl.pallas_call(
        _gemm_body,
        grid_spec=grid_spec,
        out_shape=jax.ShapeDtypeStruct((M, N), jnp.float32),
        compiler_params=pltpu.CompilerParams(
            dimension_semantics=("arbitrary", "arbitrary"),
        ),
    )(meta, x_g, weights, bias.reshape(E, 1, N))


# speedup vs baseline: 1.6094x; 1.6094x over previous
import functools

import jax
import jax.numpy as jnp
from jax import lax
from jax.experimental import pallas as pl
from jax.experimental.pallas import tpu as pltpu
from jax.experimental.pallas import tpu_sc as plsc

TM = 256
TN = 4096
CHUNK = 32


def _sc_scatter(x, si0, si1, M):
    ntokens, K = x.shape
    info = plsc.get_sparse_core_info()
    NW = info.num_cores * info.num_subcores
    tpw = ntokens // NW
    nchunks = tpw // CHUNK
    mesh = plsc.VectorSubcoreMesh(core_axis_name="c", subcore_axis_name="s")

    @functools.partial(
        pl.kernel,
        mesh=mesh,
        out_type=jax.ShapeDtypeStruct((M, K), jnp.float32),
        scratch_types=[
            pltpu.VMEM((CHUNK, K), jnp.float32),
            pltpu.VMEM((CHUNK,), jnp.int32),
            pltpu.VMEM((CHUNK,), jnp.int32),
            pltpu.SemaphoreType.DMA,
        ],
    )
    def scatter_k(x_hbm, si0_hbm, si1_hbm, out_hbm, rows_v, i0_v, i1_v, sem):
        wid = lax.axis_index("s") * info.num_cores + lax.axis_index("c")
        base = wid * tpw
        for c in range(nchunks):
            t0 = base + c * CHUNK
            pltpu.sync_copy(x_hbm.at[pl.ds(t0, CHUNK)], rows_v)
            pltpu.sync_copy(si0_hbm.at[pl.ds(t0, CHUNK)], i0_v)
            pltpu.sync_copy(si1_hbm.at[pl.ds(t0, CHUNK)], i1_v)
            cp0 = pltpu.async_copy(rows_v, out_hbm.at[i0_v], sem)
            cp1 = pltpu.async_copy(rows_v, out_hbm.at[i1_v], sem)
            cp0.wait()
            cp1.wait()

    return scatter_k(x, si0, si1)


def _tile_metadata(splits, M, E, G):
    NT = M // TM
    ends = jnp.cumsum(splits)
    starts = ends - splits
    t_lo = starts // TM
    t_hi = jnp.where(splits > 0, (ends - 1) // TM, t_lo)
    tiles = jnp.where(splits > 0, t_hi - t_lo + 1, 0)
    bounds = jnp.cumsum(tiles)
    p = jnp.arange(G, dtype=jnp.int32)
    e_p = jnp.sum((p[:, None] >= bounds[None, :]).astype(jnp.int32), axis=1)
    valid = e_p < E
    e_c = jnp.minimum(e_p, E - 1)
    prev = jnp.where(e_c > 0, jnp.take(bounds, e_c - 1, mode="clip"), 0)
    t_p = jnp.where(valid, jnp.take(t_lo, e_c) + (p - prev), NT - 1)
    ps = jnp.where(valid, jnp.maximum(jnp.take(starts, e_c), t_p * TM), 1)
    pe = jnp.where(valid, jnp.minimum(jnp.take(ends, e_c), (t_p + 1) * TM), 0)
    first = jnp.concatenate(
        [jnp.ones((1,), jnp.int32), (t_p[1:] != t_p[:-1]).astype(jnp.int32)]
    )
    return jnp.stack([e_c, t_p, ps, pe, first]).astype(jnp.int32)


def _gemm_body(meta_ref, x_ref, w_ref, b_ref, out_ref):
    p = pl.program_id(1)
    t = meta_ref[1, p]
    ps = meta_ref[2, p]
    pe = meta_ref[3, p]
    first = meta_ref[4, p]

    def contrib():
        acc = lax.dot_general(
            x_ref[...].astype(jnp.bfloat16), w_ref[0].astype(jnp.bfloat16),
            (((1,), (1,)), ((), ())),
            preferred_element_type=jnp.float32,
        )
        acc = acc + b_ref[0]
        rows = t * TM + lax.broadcasted_iota(jnp.int32, (TM, 1), 0)
        return jnp.where((rows >= ps) & (rows < pe), acc, 0.0)

    @pl.when(first == 1)
    def _():
        out_ref[...] = contrib()

    @pl.when((first == 0) & (pe > ps))
    def _():
        out_ref[...] = out_ref[...] + contrib()


def kernel(input, weights, bias, splits_gpu, scatter_index):
    ntokens, topk = scatter_index.shape
    K = input.shape[1]
    E, N, _ = weights.shape
    M = ntokens * topk
    G = M // TM + E - 1

    si = scatter_index.astype(jnp.int32)
    x_g = _sc_scatter(input, si[:, 0], si[:, 1], M)

    meta = _tile_metadata(splits_gpu.astype(jnp.int32), M, E, G)

    grid_spec = pltpu.PrefetchScalarGridSpec(
        num_scalar_prefetch=1,
        grid=(N // TN, G),
        in_specs=[
            pl.BlockSpec((TM, K), lambda n, p, m: (m[1, p], 0)),
            pl.BlockSpec((1, TN, K), lambda n, p, m: (m[0, p], n, 0)),
            pl.BlockSpec((1, 1, TN), lambda n, p, m: (m[0, p], 0, n)),
        ],
        out_specs=pl.BlockSpec((TM, TN), lambda n, p, m: (m[1, p], n)),
    )
    return pl.pallas_call(
        _gemm_body,
        grid_spec=grid_spec,
        out_shape=jax.ShapeDtypeStruct((M, N), jnp.float32),
        compiler_params=pltpu.CompilerParams(
            dimension_semantics=("arbitrary", "arbitrary"),
        ),
    )(meta, x_g, weights, bias.reshape(E, 1, N))


# speedup vs baseline: 1.6456x; 1.0225x over previous
import functools

import jax
import jax.numpy as jnp
from jax import lax
from jax.experimental import pallas as pl
from jax.experimental.pallas import tpu as pltpu
from jax.experimental.pallas import tpu_sc as plsc

TM = 256
TN = 4096
CHUNK = 32


def _sc_scatter(x, si0, si1, M):
    ntokens, K = x.shape
    info = plsc.get_sparse_core_info()
    NW = info.num_cores * info.num_subcores
    tpw = ntokens // NW
    mesh = plsc.VectorSubcoreMesh(core_axis_name="c", subcore_axis_name="s")

    nchunks = tpw // CHUNK
    si0_r = si0.reshape(NW, nchunks, CHUNK)
    si1_r = si1.reshape(NW, nchunks, CHUNK)

    @functools.partial(
        pl.kernel,
        mesh=mesh,
        out_type=jax.ShapeDtypeStruct((M, K), jnp.float32),
        scratch_types=[
            pltpu.VMEM((nchunks, CHUNK), jnp.int32),
            pltpu.VMEM((nchunks, CHUNK), jnp.int32),
            pltpu.VMEM((2, CHUNK, K), jnp.float32),
            pltpu.SemaphoreType.DMA,
            pltpu.SemaphoreType.DMA,
            pltpu.SemaphoreType.DMA,
        ],
    )
    def scatter_k(x_hbm, si0_hbm, si1_hbm, out_hbm,
                  i0_v, i1_v, rows_v, sem_ld, sem_s0, sem_s1):
        wid = lax.axis_index("s") * info.num_cores + lax.axis_index("c")
        base = wid * tpw
        pltpu.sync_copy(si0_hbm.at[wid], i0_v)
        pltpu.sync_copy(si1_hbm.at[wid], i1_v)
        sems = (sem_s0, sem_s1)
        loads = {0: pltpu.async_copy(
            x_hbm.at[pl.ds(base, CHUNK)], rows_v.at[0], sem_ld)}
        pend = {}
        for c in range(nchunks):
            b = c % 2
            loads.pop(c).wait()
            if c + 1 < nchunks:
                for cp in pend.pop(c - 1, ()):
                    cp.wait()
                loads[c + 1] = pltpu.async_copy(
                    x_hbm.at[pl.ds(base + (c + 1) * CHUNK, CHUNK)],
                    rows_v.at[(c + 1) % 2], sem_ld)
            pend[c] = (
                pltpu.async_copy(rows_v.at[b], out_hbm.at[i0_v.at[c]], sems[b]),
                pltpu.async_copy(rows_v.at[b], out_hbm.at[i1_v.at[c]], sems[b]),
            )
        for cps in pend.values():
            for cp in cps:
                cp.wait()

    return scatter_k(x, si0_r, si1_r)


def _tile_metadata(splits, M, E, G):
    NT = M // TM
    ends = jnp.cumsum(splits)
    starts = ends - splits
    t_lo = starts // TM
    t_hi = jnp.where(splits > 0, (ends - 1) // TM, t_lo)
    tiles = jnp.where(splits > 0, t_hi - t_lo + 1, 0)
    bounds = jnp.cumsum(tiles)
    p = jnp.arange(G, dtype=jnp.int32)
    e_p = jnp.sum((p[:, None] >= bounds[None, :]).astype(jnp.int32), axis=1)
    valid = e_p < E
    e_c = jnp.minimum(e_p, E - 1)
    prev = jnp.where(e_c > 0, jnp.take(bounds, e_c - 1, mode="clip"), 0)
    t_p = jnp.where(valid, jnp.take(t_lo, e_c) + (p - prev), NT - 1)
    ps = jnp.where(valid, jnp.maximum(jnp.take(starts, e_c), t_p * TM), 1)
    pe = jnp.where(valid, jnp.minimum(jnp.take(ends, e_c), (t_p + 1) * TM), 0)
    first = jnp.concatenate(
        [jnp.ones((1,), jnp.int32), (t_p[1:] != t_p[:-1]).astype(jnp.int32)]
    )
    return jnp.stack([e_c, t_p, ps, pe, first]).astype(jnp.int32)


def _gemm_body(meta_ref, x_ref, w_ref, b_ref, out_ref):
    p = pl.program_id(1)
    t = meta_ref[1, p]
    ps = meta_ref[2, p]
    pe = meta_ref[3, p]
    first = meta_ref[4, p]

    interior = (ps <= t * TM) & (pe >= (t + 1) * TM)

    def acc():
        return lax.dot_general(
            x_ref[...], w_ref[0],
            (((1,), (1,)), ((), ())),
            preferred_element_type=jnp.float32,
        ) + b_ref[0]

    def contrib():
        rows = t * TM + lax.broadcasted_iota(jnp.int32, (TM, 1), 0)
        return jnp.where((rows >= ps) & (rows < pe), acc(), 0.0)

    @pl.when((first == 1) & interior)
    def _():
        out_ref[...] = acc()

    @pl.when((first == 1) & ~interior)
    def _():
        out_ref[...] = contrib()

    @pl.when((first == 0) & (pe > ps))
    def _():
        out_ref[...] = out_ref[...] + contrib()


def kernel(input, weights, bias, splits_gpu, scatter_index):
    ntokens, topk = scatter_index.shape
    K = input.shape[1]
    E, N, _ = weights.shape
    M = ntokens * topk
    G = M // TM + E - 1

    si = scatter_index.astype(jnp.int32)
    x_g = _sc_scatter(input, si[:, 0], si[:, 1], M)

    meta = _tile_metadata(splits_gpu.astype(jnp.int32), M, E, G)

    grid_spec = pltpu.PrefetchScalarGridSpec(
        num_scalar_prefetch=1,
        grid=(N // TN, G),
        in_specs=[
            pl.BlockSpec((TM, K), lambda n, p, m: (m[1, p], 0)),
            pl.BlockSpec((1, TN, K), lambda n, p, m: (m[0, p], n, 0)),
            pl.BlockSpec((1, 1, TN), lambda n, p, m: (m[0, p], 0, n)),
        ],
        out_specs=pl.BlockSpec((TM, TN), lambda n, p, m: (m[1, p], n)),
    )
    return pl.pallas_call(
        _gemm_body,
        grid_spec=grid_spec,
        out_shape=jax.ShapeDtypeStruct((M, N), jnp.float32),
        compiler_params=pltpu.CompilerParams(
            dimension_semantics=("arbitrary", "arbitrary"),
        ),
    )(meta, x_g, weights, bias.reshape(E, 1, N))
